# baseline (speedup 1.0000x reference)
import functools
import math

import jax
import jax.numpy as jnp
from jax import lax
from jax.experimental import pallas as pl
from jax.experimental.pallas import tpu as pltpu
from jax.experimental.pallas import tpu_sc as plsc

E = 8
TOP_K = 2
D_MODEL = 1024
N_FF = 4096
CAPACITY_FACTOR = 1.25

_SQRT_2_OVER_PI = math.sqrt(2.0 / math.pi)


def _gelu_tanh(x):
    return 0.5 * x * (1.0 + jnp.tanh(_SQRT_2_OVER_PI * (x + 0.044715 * x * x * x)))


def _router_body(cap, x_ref, wr_ref, dstd_ref, dstc_ref, gate_ref, counts_ref):
    bt = x_ref.shape[0]
    landfill = E * cap

    @pl.when(pl.program_id(0) == 0)
    def _():
        counts_ref[...] = jnp.zeros_like(counts_ref)

    x = x_ref[...]
    logits = lax.dot_general(x, wr_ref[...], (((1,), (1,)), ((), ())),
                             preferred_element_type=jnp.float32)

    iota_e = lax.broadcasted_iota(jnp.int32, (bt, E), 1)
    v1 = jnp.max(logits, axis=-1, keepdims=True)
    a1 = jnp.min(jnp.where(logits == v1, iota_e, E), axis=-1, keepdims=True)
    oh1 = (iota_e == a1)
    l2 = jnp.where(oh1, -jnp.inf, logits)
    v2 = jnp.max(l2, axis=-1, keepdims=True)
    a2 = jnp.min(jnp.where(l2 == v2, iota_e, E), axis=-1, keepdims=True)
    oh2 = (iota_e == a2)

    g1 = 1.0 / (1.0 + jnp.exp(v2 - v1))
    g2 = 1.0 / (1.0 + jnp.exp(v1 - v2))

    moh = (oh1 | oh2).astype(jnp.float32)
    r_iota = lax.broadcasted_iota(jnp.int32, (bt, bt), 0)
    c_iota = lax.broadcasted_iota(jnp.int32, (bt, bt), 1)
    ltri = (c_iota < r_iota).astype(jnp.float32)
    excl = lax.dot_general(ltri, moh, (((1,), (0,)), ((), ())),
                           preferred_element_type=jnp.float32)
    counts = counts_ref[0:1, 0:E]
    pos = excl + counts
    counts_ref[0:1, 0:E] = counts + jnp.sum(moh, axis=0, keepdims=True)

    capf = jnp.float32(cap)

    def slot(oh, a, g):
        p = jnp.sum(jnp.where(oh, pos, 0.0), axis=-1, keepdims=True)
        kept = p < capf
        p_i = p.astype(jnp.int32)
        e_i = a.astype(jnp.int32)
        dst_d = jnp.where(kept, e_i * cap + p_i, landfill)
        dst_c = e_i * cap + jnp.minimum(p_i, cap - 1)
        gate = jnp.where(kept, g, 0.0)
        return dst_d, dst_c, gate

    d1, c1, w1 = slot(oh1, a1, g1)
    d2, c2, w2 = slot(oh2, a2, g2)

    dstd_ref[...] = jnp.concatenate([d1.T, d2.T], axis=0)
    dstc_ref[...] = jnp.concatenate([c1.T, c2.T], axis=0)
    gate_ref[...] = jnp.concatenate(
        [jnp.broadcast_to(w1, (bt, 16))[None],
         jnp.broadcast_to(w2, (bt, 16))[None]], axis=0)


def _router_call(flat, wr, cap, interpret=False):
    n, d = flat.shape
    bt = 512
    grid = (n // bt,)
    return pl.pallas_call(
        functools.partial(_router_body, cap),
        grid=grid,
        in_specs=[
            pl.BlockSpec((bt, d), lambda i: (i, 0)),
            pl.BlockSpec((E, d), lambda i: (0, 0)),
        ],
        out_specs=[
            pl.BlockSpec((2, bt), lambda i: (0, i)),
            pl.BlockSpec((2, bt), lambda i: (0, i)),
            pl.BlockSpec((2, bt, 16), lambda i: (0, i, 0)),
        ],
        out_shape=[
            jax.ShapeDtypeStruct((2, n), jnp.int32),
            jax.ShapeDtypeStruct((2, n), jnp.int32),
            jax.ShapeDtypeStruct((2, n, 16), jnp.float32),
        ],
        scratch_shapes=[pltpu.VMEM((8, 128), jnp.float32)],
        interpret=interpret,
    )(flat, wr)


def _dispatch_call(flat, dstd, rows_out):
    n, d = flat.shape
    nc, ns = 2, 16
    nw = nc * ns
    per = n // nw
    sub = 64
    nsub = per // sub
    mesh = plsc.VectorSubcoreMesh(core_axis_name="c", subcore_axis_name="s")

    @functools.partial(
        pl.kernel,
        out_type=jax.ShapeDtypeStruct((rows_out, d), jnp.float32),
        mesh=mesh,
        scratch_types=[
            pltpu.VMEM((sub, d), jnp.float32),
            pltpu.VMEM((sub,), jnp.int32),
            pltpu.VMEM((sub,), jnp.int32),
            pltpu.SemaphoreType.DMA,
        ],
    )
    def k(flat_hbm, dstd_hbm, xall_hbm, data_v, idx0_v, idx1_v, sem):
        wid = lax.axis_index("s") * nc + lax.axis_index("c")
        base = wid * per
        for c in range(nsub):
            b = base + c * sub
            pltpu.sync_copy(dstd_hbm.at[0, pl.ds(b, sub)], idx0_v)
            pltpu.sync_copy(dstd_hbm.at[1, pl.ds(b, sub)], idx1_v)
            pltpu.sync_copy(flat_hbm.at[pl.ds(b, sub)], data_v)
            cp0 = pltpu.async_copy(data_v, xall_hbm.at[idx0_v], sem)
            cp1 = pltpu.async_copy(data_v, xall_hbm.at[idx1_v], sem)
            cp0.wait()
            cp1.wait()

    return k(flat, dstd)


def _ffn_body(nfc, x_ref, w1_ref, b1_ref, w2_ref, b2_ref, out_ref,
              hb_ref, w2b_ref, xb_ref):
    fc = pl.program_id(1)

    @pl.when(fc == 0)
    def _():
        xb_ref[...] = x_ref[...].astype(jnp.bfloat16)

    h = lax.dot_general(xb_ref[...], w1_ref[0].astype(jnp.bfloat16),
                        (((1,), (1,)), ((), ())),
                        preferred_element_type=jnp.float32)
    hb_ref[fc] = _gelu_tanh(h.astype(jnp.bfloat16)
                            + b1_ref[0].astype(jnp.bfloat16))
    w2b_ref[fc] = w2_ref[0].astype(jnp.bfloat16)

    @pl.when(fc == nfc - 1)
    def _():
        o = b2_ref[0]
        for k in range(nfc):
            o = o + lax.dot_general(hb_ref[k], w2b_ref[k],
                                    (((1,), (1,)), ((), ())),
                                    preferred_element_type=jnp.float32)
        out_ref[...] = o


def _ffn_call(x_all, w1, b1, w2, b2, cap, interpret=False):
    d = x_all.shape[1]
    f = w1.shape[1]
    fb = 512
    nfc = f // fb
    grid = (E, nfc)
    return pl.pallas_call(
        functools.partial(_ffn_body, nfc),
        grid=grid,
        in_specs=[
            pl.BlockSpec((cap, d), lambda e, fc: (e, 0)),
            pl.BlockSpec((1, fb, d), lambda e, fc: (e, fc, 0)),
            pl.BlockSpec((1, 1, fb), lambda e, fc: (e, 0, fc)),
            pl.BlockSpec((1, d, fb), lambda e, fc: (e, 0, fc)),
            pl.BlockSpec((1, 1, d), lambda e, fc: (e, 0, 0)),
        ],
        out_specs=pl.BlockSpec((cap, d), lambda e, fc: (e, 0)),
        out_shape=jax.ShapeDtypeStruct((E * cap, d), jnp.float32),
        scratch_shapes=[pltpu.VMEM((nfc, cap, fb), jnp.bfloat16),
                        pltpu.VMEM((nfc, d, fb), jnp.bfloat16),
                        pltpu.VMEM((cap, d), jnp.bfloat16)],
        interpret=interpret,
    )(x_all, w1, b1.reshape(E, 1, f), w2, b2.reshape(E, 1, d))


def _combine_call(out_all, dstc, gates, n):
    d = out_all.shape[1]
    nc, ns = 2, 16
    nw = nc * ns
    per = n // nw
    sub = 32
    nsub = per // sub
    lanes = d // 16
    mesh = plsc.VectorSubcoreMesh(core_axis_name="c", subcore_axis_name="s")

    @functools.partial(
        pl.kernel,
        out_type=jax.ShapeDtypeStruct((n, d), jnp.float32),
        mesh=mesh,
        scratch_types=[
            pltpu.VMEM((sub,), jnp.int32),
            pltpu.VMEM((sub,), jnp.int32),
            pltpu.VMEM((sub, 16), jnp.float32),
            pltpu.VMEM((sub, 16), jnp.float32),
            pltpu.VMEM((sub, d), jnp.float32),
            pltpu.VMEM((sub, d), jnp.float32),
            pltpu.VMEM((sub, d), jnp.float32),
            pltpu.SemaphoreType.DMA,
        ],
    )
    def k(out_hbm, dstc_hbm, gate_hbm, fin_hbm,
          ic0_v, ic1_v, wc0_v, wc1_v, g0_v, g1_v, f_v, sem):
        wid = lax.axis_index("s") * nc + lax.axis_index("c")
        base = wid * per
        for c in range(nsub):
            b = base + c * sub
            pltpu.sync_copy(dstc_hbm.at[0, pl.ds(b, sub)], ic0_v)
            pltpu.sync_copy(dstc_hbm.at[1, pl.ds(b, sub)], ic1_v)
            pltpu.sync_copy(gate_hbm.at[0, pl.ds(b, sub)], wc0_v)
            pltpu.sync_copy(gate_hbm.at[1, pl.ds(b, sub)], wc1_v)
            cp0 = pltpu.async_copy(out_hbm.at[ic0_v], g0_v, sem)
            cp1 = pltpu.async_copy(out_hbm.at[ic1_v], g1_v, sem)
            cp0.wait()
            cp1.wait()

            def row(i, _):
                wv0 = wc0_v[i]
                wv1 = wc1_v[i]

                def grp(j, _):
                    s = pl.ds(j * 16, 16)
                    f_v[i, s] = wv0 * g0_v[i, s] + wv1 * g1_v[i, s]
                    return 0

                lax.fori_loop(0, lanes, grp, 0)
                return 0

            lax.fori_loop(0, sub, row, 0)
            pltpu.sync_copy(f_v, fin_hbm.at[pl.ds(b, sub)])

    return k(out_all, dstc, gates)


def kernel(hidden_states, Wr, Wn, W1, b1, W2, b2):
    b, s, d = hidden_states.shape
    n = b * s
    flat = hidden_states.reshape(n, d)
    cap = max(8, int(((n * TOP_K) // E) * CAPACITY_FACTOR))

    dstd, dstc, gates = _router_call(flat, Wr, cap)
    x_all = _dispatch_call(flat, dstd, E * cap + 8)
    out_all = _ffn_call(x_all, W1, b1, W2, b2, cap)
    final = _combine_call(out_all, dstc, gates, n)
    return final.reshape(b, s, d)


# speedup vs baseline: 1.1039x; 1.1039x over previous
import functools
import math

import jax
import jax.numpy as jnp
from jax import lax
from jax.experimental import pallas as pl
from jax.experimental.pallas import tpu as pltpu
from jax.experimental.pallas import tpu_sc as plsc

E = 8
TOP_K = 2
D_MODEL = 1024
N_FF = 4096
CAPACITY_FACTOR = 1.25

_SQRT_2_OVER_PI = math.sqrt(2.0 / math.pi)


def _gelu_tanh(x):
    return 0.5 * x * (1.0 + jnp.tanh(_SQRT_2_OVER_PI * (x + 0.044715 * x * x * x)))


def _router_body(cap, x_ref, wr_ref, dstd_ref, dstc_ref, gate_ref, counts_ref):
    bt = x_ref.shape[0]
    landfill = E * cap

    @pl.when(pl.program_id(0) == 0)
    def _():
        counts_ref[...] = jnp.zeros_like(counts_ref)

    x = x_ref[...]
    logits = lax.dot_general(x, wr_ref[...], (((1,), (1,)), ((), ())),
                             preferred_element_type=jnp.float32)

    iota_e = lax.broadcasted_iota(jnp.int32, (bt, E), 1)
    v1 = jnp.max(logits, axis=-1, keepdims=True)
    a1 = jnp.min(jnp.where(logits == v1, iota_e, E), axis=-1, keepdims=True)
    oh1 = (iota_e == a1)
    l2 = jnp.where(oh1, -jnp.inf, logits)
    v2 = jnp.max(l2, axis=-1, keepdims=True)
    a2 = jnp.min(jnp.where(l2 == v2, iota_e, E), axis=-1, keepdims=True)
    oh2 = (iota_e == a2)

    g1 = 1.0 / (1.0 + jnp.exp(v2 - v1))
    g2 = 1.0 / (1.0 + jnp.exp(v1 - v2))

    moh = (oh1 | oh2).astype(jnp.float32)
    r_iota = lax.broadcasted_iota(jnp.int32, (bt, bt), 0)
    c_iota = lax.broadcasted_iota(jnp.int32, (bt, bt), 1)
    ltri = (c_iota < r_iota).astype(jnp.float32)
    excl = lax.dot_general(ltri, moh, (((1,), (0,)), ((), ())),
                           preferred_element_type=jnp.float32)
    counts = counts_ref[0:1, 0:E]
    pos = excl + counts
    counts_ref[0:1, 0:E] = counts + jnp.sum(moh, axis=0, keepdims=True)

    capf = jnp.float32(cap)

    def slot(oh, a, g):
        p = jnp.sum(jnp.where(oh, pos, 0.0), axis=-1, keepdims=True)
        kept = p < capf
        p_i = p.astype(jnp.int32)
        e_i = a.astype(jnp.int32)
        dst_d = jnp.where(kept, e_i * cap + p_i, landfill)
        dst_c = e_i * cap + jnp.minimum(p_i, cap - 1)
        gate = jnp.where(kept, g, 0.0)
        return dst_d, dst_c, gate

    d1, c1, w1 = slot(oh1, a1, g1)
    d2, c2, w2 = slot(oh2, a2, g2)

    dstd_ref[...] = jnp.concatenate([d1.T, d2.T], axis=0)
    dstc_ref[...] = jnp.concatenate([c1.T, c2.T], axis=0)
    gate_ref[...] = jnp.concatenate(
        [jnp.broadcast_to(w1, (bt, 16))[None],
         jnp.broadcast_to(w2, (bt, 16))[None]], axis=0)


def _router_call(flat, wr, cap, interpret=False):
    n, d = flat.shape
    bt = 512
    grid = (n // bt,)
    return pl.pallas_call(
        functools.partial(_router_body, cap),
        grid=grid,
        in_specs=[
            pl.BlockSpec((bt, d), lambda i: (i, 0)),
            pl.BlockSpec((E, d), lambda i: (0, 0)),
        ],
        out_specs=[
            pl.BlockSpec((2, bt), lambda i: (0, i)),
            pl.BlockSpec((2, bt), lambda i: (0, i)),
            pl.BlockSpec((2, bt, 16), lambda i: (0, i, 0)),
        ],
        out_shape=[
            jax.ShapeDtypeStruct((2, n), jnp.int32),
            jax.ShapeDtypeStruct((2, n), jnp.int32),
            jax.ShapeDtypeStruct((2, n, 16), jnp.float32),
        ],
        scratch_shapes=[pltpu.VMEM((8, 128), jnp.float32)],
        interpret=interpret,
    )(flat, wr)


def _dispatch_call(flat, dstd, rows_out):
    n, d = flat.shape
    nc, ns = 2, 16
    nw = nc * ns
    per = n // nw
    sub = 64
    nsub = per // sub
    mesh = plsc.VectorSubcoreMesh(core_axis_name="c", subcore_axis_name="s")

    @functools.partial(
        pl.kernel,
        out_type=jax.ShapeDtypeStruct((rows_out, d), jnp.float32),
        mesh=mesh,
        scratch_types=[
            pltpu.VMEM((sub, d), jnp.float32),
            pltpu.VMEM((sub,), jnp.int32),
            pltpu.VMEM((sub,), jnp.int32),
            pltpu.SemaphoreType.DMA,
        ],
    )
    def k(flat_hbm, dstd_hbm, xall_hbm, data_v, idx0_v, idx1_v, sem):
        wid = lax.axis_index("s") * nc + lax.axis_index("c")
        base = wid * per
        for c in range(nsub):
            b = base + c * sub
            pltpu.sync_copy(dstd_hbm.at[0, pl.ds(b, sub)], idx0_v)
            pltpu.sync_copy(dstd_hbm.at[1, pl.ds(b, sub)], idx1_v)
            pltpu.sync_copy(flat_hbm.at[pl.ds(b, sub)], data_v)
            cp0 = pltpu.async_copy(data_v, xall_hbm.at[idx0_v], sem)
            cp1 = pltpu.async_copy(data_v, xall_hbm.at[idx1_v], sem)
            cp0.wait()
            cp1.wait()

    return k(flat, dstd)


def _ffn_body(nfc, x_ref, w1_ref, b1_ref, w2_ref, b2_ref, out_ref,
              acc_ref, xb_ref):
    fc = pl.program_id(1)

    @pl.when(fc == 0)
    def _():
        xb_ref[...] = x_ref[...].astype(jnp.bfloat16)

    h = lax.dot_general(xb_ref[...], w1_ref[0].astype(jnp.bfloat16),
                        (((1,), (1,)), ((), ())),
                        preferred_element_type=jnp.float32)
    h = _gelu_tanh(h.astype(jnp.bfloat16) + b1_ref[0].astype(jnp.bfloat16))
    part = lax.dot_general(h, w2_ref[0].astype(jnp.bfloat16),
                           (((1,), (1,)), ((), ())),
                           preferred_element_type=jnp.float32)

    @pl.when(fc == 0)
    def _():
        acc_ref[...] = part

    @pl.when(fc > 0)
    def _():
        acc_ref[...] += part

    @pl.when(fc == nfc - 1)
    def _():
        out_ref[...] = acc_ref[...] + b2_ref[0]


def _ffn_call(x_all, w1, b1, w2, b2, cap, interpret=False):
    d = x_all.shape[1]
    f = w1.shape[1]
    fb = 1024
    nfc = f // fb
    grid = (E, nfc)
    return pl.pallas_call(
        functools.partial(_ffn_body, nfc),
        grid=grid,
        in_specs=[
            pl.BlockSpec((cap, d), lambda e, fc: (e, 0)),
            pl.BlockSpec((1, fb, d), lambda e, fc: (e, fc, 0)),
            pl.BlockSpec((1, 1, fb), lambda e, fc: (e, 0, fc)),
            pl.BlockSpec((1, d, fb), lambda e, fc: (e, 0, fc)),
            pl.BlockSpec((1, 1, d), lambda e, fc: (e, 0, 0)),
        ],
        out_specs=pl.BlockSpec((cap, d), lambda e, fc: (e, 0)),
        out_shape=jax.ShapeDtypeStruct((E * cap, d), jnp.float32),
        scratch_shapes=[pltpu.VMEM((cap, d), jnp.float32),
                        pltpu.VMEM((cap, d), jnp.bfloat16)],
        interpret=interpret,
    )(x_all, w1, b1.reshape(E, 1, f), w2, b2.reshape(E, 1, d))


def _combine_call(out_all, dstc, gates, n):
    d = out_all.shape[1]
    nc, ns = 2, 16
    nw = nc * ns
    per = n // nw
    sub = 32
    nsub = per // sub
    lanes = d // 16
    mesh = plsc.VectorSubcoreMesh(core_axis_name="c", subcore_axis_name="s")

    @functools.partial(
        pl.kernel,
        out_type=jax.ShapeDtypeStruct((n, d), jnp.float32),
        mesh=mesh,
        scratch_types=[
            pltpu.VMEM((sub,), jnp.int32),
            pltpu.VMEM((sub,), jnp.int32),
            pltpu.VMEM((sub, 16), jnp.float32),
            pltpu.VMEM((sub, 16), jnp.float32),
            pltpu.VMEM((sub, d), jnp.float32),
            pltpu.VMEM((sub, d), jnp.float32),
            pltpu.VMEM((sub, d), jnp.float32),
            pltpu.SemaphoreType.DMA,
        ],
    )
    def k(out_hbm, dstc_hbm, gate_hbm, fin_hbm,
          ic0_v, ic1_v, wc0_v, wc1_v, g0_v, g1_v, f_v, sem):
        wid = lax.axis_index("s") * nc + lax.axis_index("c")
        base = wid * per
        for c in range(nsub):
            b = base + c * sub
            pltpu.sync_copy(dstc_hbm.at[0, pl.ds(b, sub)], ic0_v)
            pltpu.sync_copy(dstc_hbm.at[1, pl.ds(b, sub)], ic1_v)
            pltpu.sync_copy(gate_hbm.at[0, pl.ds(b, sub)], wc0_v)
            pltpu.sync_copy(gate_hbm.at[1, pl.ds(b, sub)], wc1_v)
            cp0 = pltpu.async_copy(out_hbm.at[ic0_v], g0_v, sem)
            cp1 = pltpu.async_copy(out_hbm.at[ic1_v], g1_v, sem)
            cp0.wait()
            cp1.wait()

            def row(i, _):
                wv0 = wc0_v[i]
                wv1 = wc1_v[i]

                def grp(j, _):
                    s = pl.ds(j * 16, 16)
                    f_v[i, s] = wv0 * g0_v[i, s] + wv1 * g1_v[i, s]
                    return 0

                lax.fori_loop(0, lanes, grp, 0)
                return 0

            lax.fori_loop(0, sub, row, 0)
            pltpu.sync_copy(f_v, fin_hbm.at[pl.ds(b, sub)])

    return k(out_all, dstc, gates)


def kernel(hidden_states, Wr, Wn, W1, b1, W2, b2):
    b, s, d = hidden_states.shape
    n = b * s
    flat = hidden_states.reshape(n, d)
    cap = max(8, int(((n * TOP_K) // E) * CAPACITY_FACTOR))

    dstd, dstc, gates = _router_call(flat, Wr, cap)
    x_all = _dispatch_call(flat, dstd, E * cap + 8)
    out_all = _ffn_call(x_all, W1, b1, W2, b2, cap)
    final = _combine_call(out_all, dstc, gates, n)
    return final.reshape(b, s, d)


# speedup vs baseline: 1.1469x; 1.0390x over previous
import functools
import math

import jax
import jax.numpy as jnp
from jax import lax
from jax.experimental import pallas as pl
from jax.experimental.pallas import tpu as pltpu
from jax.experimental.pallas import tpu_sc as plsc

E = 8
TOP_K = 2
D_MODEL = 1024
N_FF = 4096
CAPACITY_FACTOR = 1.25

_SQRT_2_OVER_PI = math.sqrt(2.0 / math.pi)


def _gelu_tanh(x):
    return 0.5 * x * (1.0 + jnp.tanh(_SQRT_2_OVER_PI * (x + 0.044715 * x * x * x)))


def _router_body(cap, x_ref, wr_ref, dstd_ref, dstc_ref, gate_ref, counts_ref):
    bt = x_ref.shape[0]
    landfill = E * cap

    @pl.when(pl.program_id(0) == 0)
    def _():
        counts_ref[...] = jnp.zeros_like(counts_ref)

    x = x_ref[...]
    logits = lax.dot_general(x, wr_ref[...], (((1,), (1,)), ((), ())),
                             preferred_element_type=jnp.float32)

    iota_e = lax.broadcasted_iota(jnp.int32, (bt, E), 1)
    v1 = jnp.max(logits, axis=-1, keepdims=True)
    a1 = jnp.min(jnp.where(logits == v1, iota_e, E), axis=-1, keepdims=True)
    oh1 = (iota_e == a1)
    l2 = jnp.where(oh1, -jnp.inf, logits)
    v2 = jnp.max(l2, axis=-1, keepdims=True)
    a2 = jnp.min(jnp.where(l2 == v2, iota_e, E), axis=-1, keepdims=True)
    oh2 = (iota_e == a2)

    g1 = 1.0 / (1.0 + jnp.exp(v2 - v1))
    g2 = 1.0 / (1.0 + jnp.exp(v1 - v2))

    moh = (oh1 | oh2).astype(jnp.float32)
    r_iota = lax.broadcasted_iota(jnp.int32, (bt, bt), 0)
    c_iota = lax.broadcasted_iota(jnp.int32, (bt, bt), 1)
    ltri = (c_iota < r_iota).astype(jnp.float32)
    excl = lax.dot_general(ltri, moh, (((1,), (0,)), ((), ())),
                           preferred_element_type=jnp.float32)
    counts = counts_ref[0:1, 0:E]
    pos = excl + counts
    counts_ref[0:1, 0:E] = counts + jnp.sum(moh, axis=0, keepdims=True)

    capf = jnp.float32(cap)

    def slot(oh, a, g):
        p = jnp.sum(jnp.where(oh, pos, 0.0), axis=-1, keepdims=True)
        kept = p < capf
        p_i = p.astype(jnp.int32)
        e_i = a.astype(jnp.int32)
        dst_d = jnp.where(kept, e_i * cap + p_i, landfill)
        dst_c = e_i * cap + jnp.minimum(p_i, cap - 1)
        gate = jnp.where(kept, g, 0.0)
        return dst_d, dst_c, gate

    d1, c1, w1 = slot(oh1, a1, g1)
    d2, c2, w2 = slot(oh2, a2, g2)

    dstd_ref[...] = jnp.concatenate([d1.T, d2.T], axis=0)
    dstc_ref[...] = jnp.concatenate([c1.T, c2.T], axis=0)
    gate_ref[...] = jnp.concatenate(
        [jnp.broadcast_to(w1, (bt, 16))[None],
         jnp.broadcast_to(w2, (bt, 16))[None]], axis=0)


def _router_call(flat, wr, cap, interpret=False):
    n, d = flat.shape
    bt = 512
    grid = (n // bt,)
    return pl.pallas_call(
        functools.partial(_router_body, cap),
        grid=grid,
        in_specs=[
            pl.BlockSpec((bt, d), lambda i: (i, 0)),
            pl.BlockSpec((E, d), lambda i: (0, 0)),
        ],
        out_specs=[
            pl.BlockSpec((2, bt), lambda i: (0, i)),
            pl.BlockSpec((2, bt), lambda i: (0, i)),
            pl.BlockSpec((2, bt, 16), lambda i: (0, i, 0)),
        ],
        out_shape=[
            jax.ShapeDtypeStruct((2, n), jnp.int32),
            jax.ShapeDtypeStruct((2, n), jnp.int32),
            jax.ShapeDtypeStruct((2, n, 16), jnp.float32),
        ],
        scratch_shapes=[pltpu.VMEM((8, 128), jnp.float32)],
        interpret=interpret,
    )(flat, wr)


def _dispatch_call(flat, dstd, rows_out):
    n, d = flat.shape
    nc, ns = 2, 16
    nw = nc * ns
    per = n // nw
    sub = 64
    nsub = per // sub
    mesh = plsc.VectorSubcoreMesh(core_axis_name="c", subcore_axis_name="s")

    @functools.partial(
        pl.kernel,
        out_type=jax.ShapeDtypeStruct((rows_out, d), jnp.float32),
        mesh=mesh,
        scratch_types=[
            pltpu.VMEM((sub, d), jnp.float32),
            pltpu.VMEM((sub,), jnp.int32),
            pltpu.VMEM((sub,), jnp.int32),
            pltpu.SemaphoreType.DMA,
        ],
    )
    def k(flat_hbm, dstd_hbm, xall_hbm, data_v, idx0_v, idx1_v, sem):
        wid = lax.axis_index("s") * nc + lax.axis_index("c")
        base = wid * per
        for c in range(nsub):
            b = base + c * sub
            pltpu.sync_copy(dstd_hbm.at[0, pl.ds(b, sub)], idx0_v)
            pltpu.sync_copy(dstd_hbm.at[1, pl.ds(b, sub)], idx1_v)
            pltpu.sync_copy(flat_hbm.at[pl.ds(b, sub)], data_v)
            cp0 = pltpu.async_copy(data_v, xall_hbm.at[idx0_v], sem)
            cp1 = pltpu.async_copy(data_v, xall_hbm.at[idx1_v], sem)
            cp0.wait()
            cp1.wait()

    return k(flat, dstd)


def _ffn_body(nfc, x_ref, w1_ref, b1_ref, w2_ref, b2_ref, out_ref,
              acc_ref, xb_ref):
    fc = pl.program_id(1)

    @pl.when(fc == 0)
    def _():
        xb_ref[...] = x_ref[...].astype(jnp.bfloat16)

    h = lax.dot_general(xb_ref[...], w1_ref[0].astype(jnp.bfloat16),
                        (((1,), (1,)), ((), ())),
                        preferred_element_type=jnp.float32)
    h = _gelu_tanh(h.astype(jnp.bfloat16) + b1_ref[0].astype(jnp.bfloat16))
    part = lax.dot_general(h, w2_ref[0].astype(jnp.bfloat16),
                           (((1,), (1,)), ((), ())),
                           preferred_element_type=jnp.float32)

    @pl.when(fc == 0)
    def _():
        acc_ref[...] = part

    @pl.when(fc > 0)
    def _():
        acc_ref[...] += part

    @pl.when(fc == nfc - 1)
    def _():
        out_ref[...] = acc_ref[...] + b2_ref[0]


def _ffn_call(x_all, w1, b1, w2, b2, cap, interpret=False):
    d = x_all.shape[1]
    f = w1.shape[1]
    fb = 1024
    nfc = f // fb
    grid = (E, nfc)
    return pl.pallas_call(
        functools.partial(_ffn_body, nfc),
        grid=grid,
        in_specs=[
            pl.BlockSpec((cap, d), lambda e, fc: (e, 0)),
            pl.BlockSpec((1, fb, d), lambda e, fc: (e, fc, 0)),
            pl.BlockSpec((1, 1, fb), lambda e, fc: (e, 0, fc)),
            pl.BlockSpec((1, d, fb), lambda e, fc: (e, 0, fc)),
            pl.BlockSpec((1, 1, d), lambda e, fc: (e, 0, 0)),
        ],
        out_specs=pl.BlockSpec((cap, d), lambda e, fc: (e, 0)),
        out_shape=jax.ShapeDtypeStruct((E * cap, d), jnp.float32),
        scratch_shapes=[pltpu.VMEM((cap, d), jnp.float32),
                        pltpu.VMEM((cap, d), jnp.bfloat16)],
        interpret=interpret,
    )(x_all, w1, b1.reshape(E, 1, f), w2, b2.reshape(E, 1, d))


def _combine_call(out_all, dstc, gates, n):
    d = out_all.shape[1]
    nc, ns = 2, 16
    nw = nc * ns
    per = n // nw
    sub = 32
    nsub = per // sub
    lanes = d // 16
    mesh = plsc.VectorSubcoreMesh(core_axis_name="c", subcore_axis_name="s")

    @functools.partial(
        pl.kernel,
        out_type=jax.ShapeDtypeStruct((n, d), jnp.float32),
        mesh=mesh,
        scratch_types=[
            pltpu.VMEM((sub,), jnp.int32),
            pltpu.VMEM((sub,), jnp.int32),
            pltpu.VMEM((sub, 16), jnp.float32),
            pltpu.VMEM((sub, 16), jnp.float32),
            pltpu.VMEM((sub, d), jnp.float32),
            pltpu.VMEM((sub, d), jnp.float32),
            pltpu.VMEM((sub, d), jnp.float32),
            pltpu.SemaphoreType.DMA,
        ],
    )
    def k(out_hbm, dstc_hbm, gate_hbm, fin_hbm,
          ic0_v, ic1_v, wc0_v, wc1_v, g0_v, g1_v, f_v, sem):
        wid = lax.axis_index("s") * nc + lax.axis_index("c")
        base = wid * per
        for c in range(nsub):
            b = base + c * sub
            pltpu.sync_copy(dstc_hbm.at[0, pl.ds(b, sub)], ic0_v)
            pltpu.sync_copy(dstc_hbm.at[1, pl.ds(b, sub)], ic1_v)
            pltpu.sync_copy(gate_hbm.at[0, pl.ds(b, sub)], wc0_v)
            pltpu.sync_copy(gate_hbm.at[1, pl.ds(b, sub)], wc1_v)
            cp0 = pltpu.async_copy(out_hbm.at[ic0_v], g0_v, sem)
            cp1 = pltpu.async_copy(out_hbm.at[ic1_v], g1_v, sem)
            cp0.wait()
            cp1.wait()

            def row(i, _):
                wv0 = wc0_v[i]
                wv1 = wc1_v[i]
                for j in range(lanes):
                    s = pl.ds(j * 16, 16)
                    f_v[i, s] = wv0 * g0_v[i, s] + wv1 * g1_v[i, s]
                return 0

            lax.fori_loop(0, sub, row, 0)
            pltpu.sync_copy(f_v, fin_hbm.at[pl.ds(b, sub)])

    return k(out_all, dstc, gates)


def kernel(hidden_states, Wr, Wn, W1, b1, W2, b2):
    b, s, d = hidden_states.shape
    n = b * s
    flat = hidden_states.reshape(n, d)
    cap = max(8, int(((n * TOP_K) // E) * CAPACITY_FACTOR))

    dstd, dstc, gates = _router_call(flat, Wr, cap)
    x_all = _dispatch_call(flat, dstd, E * cap + 8)
    out_all = _ffn_call(x_all, W1, b1, W2, b2, cap)
    final = _combine_call(out_all, dstc, gates, n)
    return final.reshape(b, s, d)


# speedup vs baseline: 1.1705x; 1.0206x over previous
import functools
import math

import jax
import jax.numpy as jnp
from jax import lax
from jax.experimental import pallas as pl
from jax.experimental.pallas import tpu as pltpu
from jax.experimental.pallas import tpu_sc as plsc

E = 8
TOP_K = 2
D_MODEL = 1024
N_FF = 4096
CAPACITY_FACTOR = 1.25

_SQRT_2_OVER_PI = math.sqrt(2.0 / math.pi)


def _gelu_tanh(x):
    return 0.5 * x * (1.0 + jnp.tanh(_SQRT_2_OVER_PI * (x + 0.044715 * x * x * x)))


def _router_body(cap, x_ref, wr_ref, dstd_ref, dstc_ref, gate_ref, counts_ref):
    bt = x_ref.shape[0]
    landfill = E * cap

    @pl.when(pl.program_id(0) == 0)
    def _():
        counts_ref[...] = jnp.zeros_like(counts_ref)

    x = x_ref[...]
    logits = lax.dot_general(x, wr_ref[...], (((1,), (1,)), ((), ())),
                             preferred_element_type=jnp.float32)

    iota_e = lax.broadcasted_iota(jnp.int32, (bt, E), 1)
    v1 = jnp.max(logits, axis=-1, keepdims=True)
    a1 = jnp.min(jnp.where(logits == v1, iota_e, E), axis=-1, keepdims=True)
    oh1 = (iota_e == a1)
    l2 = jnp.where(oh1, -jnp.inf, logits)
    v2 = jnp.max(l2, axis=-1, keepdims=True)
    a2 = jnp.min(jnp.where(l2 == v2, iota_e, E), axis=-1, keepdims=True)
    oh2 = (iota_e == a2)

    g1 = 1.0 / (1.0 + jnp.exp(v2 - v1))
    g2 = 1.0 / (1.0 + jnp.exp(v1 - v2))

    moh = (oh1 | oh2).astype(jnp.float32)
    r_iota = lax.broadcasted_iota(jnp.int32, (bt, bt), 0)
    c_iota = lax.broadcasted_iota(jnp.int32, (bt, bt), 1)
    ltri = (c_iota < r_iota).astype(jnp.float32)
    excl = lax.dot_general(ltri, moh, (((1,), (0,)), ((), ())),
                           preferred_element_type=jnp.float32)
    counts = counts_ref[0:1, 0:E]
    pos = excl + counts
    counts_ref[0:1, 0:E] = counts + jnp.sum(moh, axis=0, keepdims=True)

    capf = jnp.float32(cap)

    def slot(oh, a, g):
        p = jnp.sum(jnp.where(oh, pos, 0.0), axis=-1, keepdims=True)
        kept = p < capf
        p_i = p.astype(jnp.int32)
        e_i = a.astype(jnp.int32)
        dst_d = jnp.where(kept, e_i * cap + p_i, landfill)
        dst_c = e_i * cap + jnp.minimum(p_i, cap - 1)
        gate = jnp.where(kept, g, 0.0)
        return dst_d, dst_c, gate

    d1, c1, w1 = slot(oh1, a1, g1)
    d2, c2, w2 = slot(oh2, a2, g2)

    dstd_ref[...] = jnp.concatenate([d1.T, d2.T], axis=0)
    dstc_ref[...] = jnp.concatenate([c1.T, c2.T], axis=0)
    gate_ref[...] = jnp.concatenate(
        [jnp.broadcast_to(w1, (bt, 16))[None],
         jnp.broadcast_to(w2, (bt, 16))[None]], axis=0)


def _router_call(flat, wr, cap, interpret=False):
    n, d = flat.shape
    bt = 512
    grid = (n // bt,)
    return pl.pallas_call(
        functools.partial(_router_body, cap),
        grid=grid,
        in_specs=[
            pl.BlockSpec((bt, d), lambda i: (i, 0)),
            pl.BlockSpec((E, d), lambda i: (0, 0)),
        ],
        out_specs=[
            pl.BlockSpec((2, bt), lambda i: (0, i)),
            pl.BlockSpec((2, bt), lambda i: (0, i)),
            pl.BlockSpec((2, bt, 16), lambda i: (0, i, 0)),
        ],
        out_shape=[
            jax.ShapeDtypeStruct((2, n), jnp.int32),
            jax.ShapeDtypeStruct((2, n), jnp.int32),
            jax.ShapeDtypeStruct((2, n, 16), jnp.float32),
        ],
        scratch_shapes=[pltpu.VMEM((8, 128), jnp.float32)],
        interpret=interpret,
    )(flat, wr)


def _dispatch_call(flat, dstd, rows_out):
    n, d = flat.shape
    nc, ns = 2, 16
    nw = nc * ns
    per = n // nw
    sub = 64
    nsub = per // sub
    mesh = plsc.VectorSubcoreMesh(core_axis_name="c", subcore_axis_name="s")

    @functools.partial(
        pl.kernel,
        out_type=jax.ShapeDtypeStruct((rows_out, d), jnp.float32),
        mesh=mesh,
        scratch_types=[
            pltpu.VMEM((sub, d), jnp.float32),
            pltpu.VMEM((sub,), jnp.int32),
            pltpu.VMEM((sub,), jnp.int32),
            pltpu.SemaphoreType.DMA,
        ],
    )
    def k(flat_hbm, dstd_hbm, xall_hbm, data_v, idx0_v, idx1_v, sem):
        wid = lax.axis_index("s") * nc + lax.axis_index("c")
        base = wid * per
        for c in range(nsub):
            b = base + c * sub
            pltpu.sync_copy(dstd_hbm.at[0, pl.ds(b, sub)], idx0_v)
            pltpu.sync_copy(dstd_hbm.at[1, pl.ds(b, sub)], idx1_v)
            pltpu.sync_copy(flat_hbm.at[pl.ds(b, sub)], data_v)
            cp0 = pltpu.async_copy(data_v, xall_hbm.at[idx0_v], sem)
            cp1 = pltpu.async_copy(data_v, xall_hbm.at[idx1_v], sem)
            cp0.wait()
            cp1.wait()

    return k(flat, dstd)


def _ffn_body(nfc, x_ref, w1_ref, b1_ref, w2_ref, b2_ref, out_ref,
              acc_ref, xb_ref):
    fc = pl.program_id(1)

    @pl.when(fc == 0)
    def _():
        xb_ref[...] = x_ref[...].astype(jnp.bfloat16)

    h = lax.dot_general(xb_ref[...], w1_ref[0].astype(jnp.bfloat16),
                        (((1,), (1,)), ((), ())),
                        preferred_element_type=jnp.float32)
    h = _gelu_tanh(h.astype(jnp.bfloat16) + b1_ref[0].astype(jnp.bfloat16))
    part = lax.dot_general(h, w2_ref[0].astype(jnp.bfloat16),
                           (((1,), (1,)), ((), ())),
                           preferred_element_type=jnp.float32)

    @pl.when(fc == 0)
    def _():
        acc_ref[...] = part

    @pl.when(fc > 0)
    def _():
        acc_ref[...] += part

    @pl.when(fc == nfc - 1)
    def _():
        out_ref[...] = acc_ref[...] + b2_ref[0]


def _ffn_call(x_all, w1, b1, w2, b2, cap, interpret=False):
    d = x_all.shape[1]
    f = w1.shape[1]
    fb = 1024
    nfc = f // fb
    grid = (E, nfc)
    return pl.pallas_call(
        functools.partial(_ffn_body, nfc),
        grid=grid,
        in_specs=[
            pl.BlockSpec((cap, d), lambda e, fc: (e, 0)),
            pl.BlockSpec((1, fb, d), lambda e, fc: (e, fc, 0)),
            pl.BlockSpec((1, 1, fb), lambda e, fc: (e, 0, fc)),
            pl.BlockSpec((1, d, fb), lambda e, fc: (e, 0, fc)),
            pl.BlockSpec((1, 1, d), lambda e, fc: (e, 0, 0)),
        ],
        out_specs=pl.BlockSpec((cap, d), lambda e, fc: (e, 0)),
        out_shape=jax.ShapeDtypeStruct((E * cap, d), jnp.float32),
        scratch_shapes=[pltpu.VMEM((cap, d), jnp.float32),
                        pltpu.VMEM((cap, d), jnp.bfloat16)],
        interpret=interpret,
    )(x_all, w1, b1.reshape(E, 1, f), w2, b2.reshape(E, 1, d))


def _combine_call(out_all, dstc, gates, n):
    d = out_all.shape[1]
    nc, ns = 2, 16
    nw = nc * ns
    per = n // nw
    sub = 8
    nsub = per // sub
    lanes = d // 16
    mesh = plsc.VectorSubcoreMesh(core_axis_name="c", subcore_axis_name="s")

    @functools.partial(
        pl.kernel,
        out_type=jax.ShapeDtypeStruct((n, d), jnp.float32),
        mesh=mesh,
        scratch_types=[
            pltpu.VMEM((2, per), jnp.int32),
            pltpu.VMEM((2, per, 16), jnp.float32),
            pltpu.VMEM((2, sub, d), jnp.float32),
            pltpu.VMEM((2, sub, d), jnp.float32),
            pltpu.VMEM((2, sub, d), jnp.float32),
            pltpu.SemaphoreType.DMA,
            pltpu.SemaphoreType.DMA,
        ],
    )
    def k(out_hbm, dstc_hbm, gate_hbm, fin_hbm,
          ic_v, wc_v, g0_v, g1_v, f_v, gsem, ssem):
        wid = lax.axis_index("s") * nc + lax.axis_index("c")
        base = wid * per
        pltpu.sync_copy(dstc_hbm.at[0, pl.ds(base, per)], ic_v.at[0])
        pltpu.sync_copy(dstc_hbm.at[1, pl.ds(base, per)], ic_v.at[1])
        pltpu.sync_copy(gate_hbm.at[0, pl.ds(base, per)], wc_v.at[0])
        pltpu.sync_copy(gate_hbm.at[1, pl.ds(base, per)], wc_v.at[1])

        def gather(c):
            buf = c % 2
            o = pl.ds(c * sub, sub)
            cpa = pltpu.async_copy(out_hbm.at[ic_v.at[0, o]], g0_v.at[buf],
                                   gsem)
            cpb = pltpu.async_copy(out_hbm.at[ic_v.at[1, o]], g1_v.at[buf],
                                   gsem)
            return cpa, cpb

        store_cps = [None, None]
        pend = gather(0)
        for c in range(nsub):
            buf = c % 2
            pend[0].wait()
            pend[1].wait()
            if c + 1 < nsub:
                pend = gather(c + 1)
            if store_cps[buf] is not None:
                store_cps[buf].wait()

            def row(i, _, c=c, buf=buf):
                wv0 = wc_v[0, c * sub + i]
                wv1 = wc_v[1, c * sub + i]
                for j in range(lanes):
                    s = pl.ds(j * 16, 16)
                    f_v[buf, i, s] = (wv0 * g0_v[buf, i, s]
                                      + wv1 * g1_v[buf, i, s])
                return 0

            lax.fori_loop(0, sub, row, 0)
            store_cps[buf] = pltpu.async_copy(
                f_v.at[buf], fin_hbm.at[pl.ds(base + c * sub, sub)], ssem)
        store_cps[0].wait()
        store_cps[1].wait()

    return k(out_all, dstc, gates)


def kernel(hidden_states, Wr, Wn, W1, b1, W2, b2):
    b, s, d = hidden_states.shape
    n = b * s
    flat = hidden_states.reshape(n, d)
    cap = max(8, int(((n * TOP_K) // E) * CAPACITY_FACTOR))

    dstd, dstc, gates = _router_call(flat, Wr, cap)
    x_all = _dispatch_call(flat, dstd, E * cap + 8)
    out_all = _ffn_call(x_all, W1, b1, W2, b2, cap)
    final = _combine_call(out_all, dstc, gates, n)
    return final.reshape(b, s, d)
